# baseline (speedup 1.0000x reference)
import jax
import jax.numpy as jnp
from jax import lax
from jax.experimental import pallas as pl
from jax.experimental.pallas import tpu as pltpu
from jax.experimental.pallas import tpu_sc as plsc

jax.config.update("jax_enable_x64", True)

B = 1024
T = 4096
MB = 16
L = 16
NC, NS = 2, 16
NW = NC * NS
RPW = B // NW
FB_POS = 128
FB_F = FB_POS * 2
ROW_F = 2 * T


def _seg_update(t, im, imp, cnt, sm, cref, ci, ones):
    edge = im * (1 - imp)
    c = cref[ci]
    cs = plsc.cumsum(edge) + c
    label = jnp.where(jnp.logical_and(im > 0, cs < MB), cs, 0)
    plsc.addupdate_scatter(cnt, [label], ones)
    plsc.addupdate_scatter(sm, [label], t)
    cref[ci] = c + jnp.sum(edge, dtype=jnp.int32)


def _chunk_masks(vals):
    a0, a1, am, bm = vals
    im = (jnp.minimum(a0, a1) < 0.5).astype(jnp.int32)
    imp = (jnp.minimum(am, bm) < 0.5).astype(jnp.int32)
    return im, imp


def _row_scan(buf, n_chunks, cnt, sm, cref, ci, lane):
    ones = jnp.ones((L,), jnp.int32)

    def chunk(i, carry):
        @pl.when(cref[ci] < MB)
        def _():
            t = i * L + lane
            tm = jnp.maximum(t - 1, 0)
            f0 = ((t >> 7) << 8) + (t & 127)
            fm0 = ((tm >> 7) << 8) + (tm & 127)
            im, imp = _chunk_masks([plsc.load_gather(buf, [f]) for f in
                                    (f0, f0 + 128, fm0, fm0 + 128)])
            first = jnp.logical_and(lane == 0, i == 0)
            imp = jnp.where(first, 0, imp)
            _seg_update(t, im, imp, cnt, sm, cref, ci, ones)
        return carry

    lax.fori_loop(jnp.int32(0), jnp.int32(n_chunks), chunk, jnp.int32(0))


def _multi_scan(streams, n_chunks, cref, lane):
    ones = jnp.ones((L,), jnp.int32)
    UNROLL = 2

    def subchunk(t, first_flag):
        tm = jnp.maximum(t - 1, 0)
        f0 = ((t >> 7) << 8) + (t & 127)
        fm0 = ((tm >> 7) << 8) + (tm & 127)
        masks = []
        for fb, rsel, ci, cnt, sm in streams:
            im, imp = _chunk_masks([plsc.load_gather(fb, [rsel, f]) for
                                    f in (f0, f0 + 128, fm0, fm0 + 128)])
            masks.append((im, jnp.where(first_flag, 0, imp)))
        for (fb, rsel, ci, cnt, sm), (im, imp) in zip(streams, masks):
            _seg_update(t, im, imp, cnt, sm, cref, ci, ones)

    def chunk(i, carry):
        active = cref[streams[0][2]] < MB
        for _, _, ci, _, _ in streams[1:]:
            active = jnp.logical_or(active, cref[ci] < MB)

        @pl.when(active)
        def _():
            for u in range(UNROLL):
                t = (i * UNROLL + u) * L + lane
                first = (jnp.logical_and(lane == 0, i == 0) if u == 0
                         else jnp.zeros((L,), jnp.bool_))
                subchunk(t, first)
        return carry

    lax.fori_loop(jnp.int32(0), jnp.int32(n_chunks // UNROLL), chunk,
                  jnp.int32(0))


def _sc_body(t_hbm, p_hbm, out_hbm, fb_t, fb_p, rest, cnt0, sm0, cnt1, sm1,
             outv, cref, sem_a, sem_b):
    cid = lax.axis_index("c")
    sid = lax.axis_index("s")
    wid = sid * NC + cid
    base = wid * RPW
    lane = lax.iota(jnp.int32, L)
    zeros = jnp.zeros((L,), jnp.int32)
    bins = ((cnt0, sm0), (cnt1, sm1))

    cp_a = pltpu.async_copy(
        t_hbm.at[pl.ds(base, RPW), pl.ds(0, FB_F)], fb_t, sem_a)
    cp_b = pltpu.async_copy(
        p_hbm.at[pl.ds(base, RPW), pl.ds(0, FB_F)], fb_p, sem_b)
    cp_a.wait()
    cp_b.wait()

    outv[...] = jnp.zeros((L,), jnp.float32)

    def stats_from(cnt, sm):
        cntv = cnt[...]
        smv = sm[...]
        nb = jnp.max(jnp.where(jnp.logical_and(cntv > 0, lane >= 1), lane, 0))
        pos = smv.astype(jnp.float32) / jnp.maximum(cntv, 1).astype(jnp.float32)
        valid = jnp.logical_and(lane >= 1, lane <= nb)
        return nb, valid, pos

    def row_metrics(nb_t, valid_t, pos_t, nb_p, valid_p, pos_p):
        post = jnp.where(valid_t, pos_t, jnp.float32(1e9))
        dn = lax.GatherDimensionNumbers(
            offset_dims=(), collapsed_slice_dims=(0,), start_index_map=(0,))
        closest = jnp.abs(pos_p - post)
        for s in range(1, MB):
            idx = (lane + s) & (L - 1)
            pr = lax.gather(post, idx[:, None], dn, (1,),
                            mode=lax.GatherScatterMode.PROMISE_IN_BOUNDS)
            closest = jnp.minimum(closest, jnp.abs(pos_p - pr))
        radius = jnp.max(jnp.where(valid_p, closest, jnp.float32(-1.0)))
        counted = jnp.logical_and(nb_t > 0, nb_p > 0)
        r_c = jnp.where(counted, radius, jnp.float32(0.0))
        n_c = jnp.where(counted, jnp.float32(1.0), jnp.float32(0.0))
        mae_c = jnp.abs(nb_t - nb_p).astype(jnp.float32)
        contrib = (jnp.where(lane == 0, mae_c, jnp.float32(0.0))
                   + jnp.where(lane == 1, r_c, jnp.float32(0.0))
                   + jnp.where(lane == 2, n_c, jnp.float32(0.0)))
        outv[...] = outv[...] + contrib

    def row_body(r, carry):
        streams = []
        rsel = lax.broadcast_in_dim(r, (L,), ())
        for k in range(2):
            cnt, sm = bins[k]
            cnt[...] = zeros
            sm[...] = zeros
            cref[k] = jnp.int32(0)
            streams.append((fb_t if k == 0 else fb_p, rsel, k, cnt, sm))
        _multi_scan(streams, FB_POS // L, cref, lane)

        for k in range(2):
            cnt, sm = bins[k]
            hbm = t_hbm if k == 0 else p_hbm

            @pl.when(cref[k] < MB)
            def _slow(cnt=cnt, sm=sm, hbm=hbm, k=k):
                pltpu.sync_copy(hbm.at[base + r], rest)
                cnt[...] = zeros
                sm[...] = zeros
                cref[k] = jnp.int32(0)
                _row_scan(rest, T // L, cnt, sm, cref, k, lane)

        st = [stats_from(*bins[k]) for k in range(2)]
        row_metrics(*st[0], *st[1])
        return carry

    lax.fori_loop(jnp.int32(0), jnp.int32(RPW), row_body, jnp.int32(0))
    pltpu.sync_copy(outv, out_hbm.at[wid])


@jax.jit
def _run(t2d, p2d):
    mesh = plsc.VectorSubcoreMesh(
        core_axis_name="c", subcore_axis_name="s",
        num_cores=NC, num_subcores=NS)
    kern = pl.kernel(
        _sc_body,
        out_type=jax.ShapeDtypeStruct((NW, L), jnp.float32),
        mesh=mesh,
        compiler_params=pltpu.CompilerParams(
            needs_layout_passes=False, use_tc_tiling_on_sc=False),
        scratch_types=[
            pltpu.VMEM((RPW, FB_F), jnp.float32),
            pltpu.VMEM((RPW, FB_F), jnp.float32),
            pltpu.VMEM((ROW_F,), jnp.float32),
            pltpu.VMEM((L,), jnp.int32),
            pltpu.VMEM((L,), jnp.int32),
            pltpu.VMEM((L,), jnp.int32),
            pltpu.VMEM((L,), jnp.int32),
            pltpu.VMEM((L,), jnp.float32),
            pltpu.SMEM((2,), jnp.int32),
            pltpu.SemaphoreType.DMA,
            pltpu.SemaphoreType.DMA,
        ],
    )
    return kern(t2d, p2d)


def _native_view(x):
    return x.reshape(B, T // 128, 128, 2).transpose(0, 1, 3, 2).reshape(B, ROW_F)


def kernel(y_true_affinity, y_pred_affinity):
    t2d = _native_view(y_true_affinity)
    p2d = _native_view(y_pred_affinity)
    parts = _run(t2d, p2d)
    packed = jnp.sum(parts[:, :3], axis=0).astype(jnp.float64)
    n_delta = jnp.asarray(float(B), jnp.float64)
    return (packed[0], n_delta, packed[1], packed[2])


# speedup vs baseline: 1.1641x; 1.1641x over previous
import jax
import jax.numpy as jnp
from jax import lax
from jax.experimental import pallas as pl
from jax.experimental.pallas import tpu as pltpu
from jax.experimental.pallas import tpu_sc as plsc

jax.config.update("jax_enable_x64", True)

B = 1024
T = 4096
MB = 16
L = 16
NC, NS = 2, 16
NW = NC * NS
RPW = B // NW
FB_POS = 256
FB_F = FB_POS * 2
ROW_F = 2 * T


def _seg_update(t, im, imp, cnt, sm, cref, ci, ones):
    edge = im * (1 - imp)
    c = cref[ci]
    cs = plsc.cumsum(edge) + c
    label = jnp.where(jnp.logical_and(im > 0, cs < MB), cs, 0)
    plsc.addupdate_scatter(cnt, [label], ones)
    plsc.addupdate_scatter(sm, [label], t)
    cref[ci] = c + jnp.sum(edge, dtype=jnp.int32)


def _chunk_masks(vals):
    a0, a1, am, bm = vals
    im = (jnp.minimum(a0, a1) < 0.5).astype(jnp.int32)
    imp = (jnp.minimum(am, bm) < 0.5).astype(jnp.int32)
    return im, imp


def _row_scan(buf, n_chunks, cnt, sm, cref, ci, lane):
    ones = jnp.ones((L,), jnp.int32)

    def chunk(i, carry):
        @pl.when(cref[ci] < MB)
        def _():
            t = i * L + lane
            tm = jnp.maximum(t - 1, 0)
            f0 = ((t >> 7) << 8) + (t & 127)
            fm0 = ((tm >> 7) << 8) + (tm & 127)
            im, imp = _chunk_masks([plsc.load_gather(buf, [f]) for f in
                                    (f0, f0 + 128, fm0, fm0 + 128)])
            first = jnp.logical_and(lane == 0, i == 0)
            imp = jnp.where(first, 0, imp)
            _seg_update(t, im, imp, cnt, sm, cref, ci, ones)
        return carry

    lax.fori_loop(jnp.int32(0), jnp.int32(n_chunks), chunk, jnp.int32(0))


def _multi_scan(streams, n_chunks, cref, lane):
    ones = jnp.ones((L,), jnp.int32)
    UNROLL = 2

    def subchunk(t, first_flag):
        tm = jnp.maximum(t - 1, 0)
        f0 = ((t >> 7) << 8) + (t & 127)
        fm0 = ((tm >> 7) << 8) + (tm & 127)
        masks = []
        for fb, rsel, ci, cnt, sm in streams:
            im, imp = _chunk_masks([plsc.load_gather(fb, [rsel, f]) for
                                    f in (f0, f0 + 128, fm0, fm0 + 128)])
            masks.append((im, jnp.where(first_flag, 0, imp)))
        for (fb, rsel, ci, cnt, sm), (im, imp) in zip(streams, masks):
            _seg_update(t, im, imp, cnt, sm, cref, ci, ones)

    def chunk(i, carry):
        active = cref[streams[0][2]] < MB
        for _, _, ci, _, _ in streams[1:]:
            active = jnp.logical_or(active, cref[ci] < MB)

        @pl.when(active)
        def _():
            for u in range(UNROLL):
                t = (i * UNROLL + u) * L + lane
                first = (jnp.logical_and(lane == 0, i == 0) if u == 0
                         else jnp.zeros((L,), jnp.bool_))
                subchunk(t, first)
        return carry

    lax.fori_loop(jnp.int32(0), jnp.int32(n_chunks // UNROLL), chunk,
                  jnp.int32(0))


def _sc_body(t_hbm, p_hbm, out_hbm, fb_t, fb_p, rest, cnt0, sm0, cnt1, sm1,
             outv, cref, sem_a, sem_b):
    cid = lax.axis_index("c")
    sid = lax.axis_index("s")
    wid = sid * NC + cid
    base = wid * RPW
    lane = lax.iota(jnp.int32, L)
    zeros = jnp.zeros((L,), jnp.int32)
    bins = ((cnt0, sm0), (cnt1, sm1))

    cp_a = pltpu.async_copy(
        t_hbm.at[pl.ds(base, RPW), pl.ds(0, FB_F)], fb_t, sem_a)
    cp_b = pltpu.async_copy(
        p_hbm.at[pl.ds(base, RPW), pl.ds(0, FB_F)], fb_p, sem_b)
    cp_a.wait()
    cp_b.wait()

    outv[...] = jnp.zeros((L,), jnp.float32)

    def stats_from(cnt, sm):
        cntv = cnt[...]
        smv = sm[...]
        nb = jnp.max(jnp.where(jnp.logical_and(cntv > 0, lane >= 1), lane, 0))
        pos = smv.astype(jnp.float32) / jnp.maximum(cntv, 1).astype(jnp.float32)
        valid = jnp.logical_and(lane >= 1, lane <= nb)
        return nb, valid, pos

    def row_metrics(nb_t, valid_t, pos_t, nb_p, valid_p, pos_p):
        post = jnp.where(valid_t, pos_t, jnp.float32(1e9))
        dn = lax.GatherDimensionNumbers(
            offset_dims=(), collapsed_slice_dims=(0,), start_index_map=(0,))
        closest = jnp.abs(pos_p - post)
        for s in range(1, MB):
            idx = (lane + s) & (L - 1)
            pr = lax.gather(post, idx[:, None], dn, (1,),
                            mode=lax.GatherScatterMode.PROMISE_IN_BOUNDS)
            closest = jnp.minimum(closest, jnp.abs(pos_p - pr))
        radius = jnp.max(jnp.where(valid_p, closest, jnp.float32(-1.0)))
        counted = jnp.logical_and(nb_t > 0, nb_p > 0)
        r_c = jnp.where(counted, radius, jnp.float32(0.0))
        n_c = jnp.where(counted, jnp.float32(1.0), jnp.float32(0.0))
        mae_c = jnp.abs(nb_t - nb_p).astype(jnp.float32)
        contrib = (jnp.where(lane == 0, mae_c, jnp.float32(0.0))
                   + jnp.where(lane == 1, r_c, jnp.float32(0.0))
                   + jnp.where(lane == 2, n_c, jnp.float32(0.0)))
        outv[...] = outv[...] + contrib

    def row_body(r, carry):
        streams = []
        rsel = lax.broadcast_in_dim(r, (L,), ())
        for k in range(2):
            cnt, sm = bins[k]
            cnt[...] = zeros
            sm[...] = zeros
            cref[k] = jnp.int32(0)
            streams.append((fb_t if k == 0 else fb_p, rsel, k, cnt, sm))
        _multi_scan(streams, FB_POS // L, cref, lane)

        for k in range(2):
            cnt, sm = bins[k]
            hbm = t_hbm if k == 0 else p_hbm

            @pl.when(cref[k] < MB)
            def _slow(cnt=cnt, sm=sm, hbm=hbm, k=k):
                pltpu.sync_copy(hbm.at[base + r], rest)
                cnt[...] = zeros
                sm[...] = zeros
                cref[k] = jnp.int32(0)
                _row_scan(rest, T // L, cnt, sm, cref, k, lane)

        st = [stats_from(*bins[k]) for k in range(2)]
        row_metrics(*st[0], *st[1])
        return carry

    lax.fori_loop(jnp.int32(0), jnp.int32(RPW), row_body, jnp.int32(0))
    pltpu.sync_copy(outv, out_hbm.at[wid])


@jax.jit
def _run(t2d, p2d):
    mesh = plsc.VectorSubcoreMesh(
        core_axis_name="c", subcore_axis_name="s",
        num_cores=NC, num_subcores=NS)
    kern = pl.kernel(
        _sc_body,
        out_type=jax.ShapeDtypeStruct((NW, L), jnp.float32),
        mesh=mesh,
        compiler_params=pltpu.CompilerParams(
            needs_layout_passes=False, use_tc_tiling_on_sc=False),
        scratch_types=[
            pltpu.VMEM((RPW, FB_F), jnp.float32),
            pltpu.VMEM((RPW, FB_F), jnp.float32),
            pltpu.VMEM((ROW_F,), jnp.float32),
            pltpu.VMEM((L,), jnp.int32),
            pltpu.VMEM((L,), jnp.int32),
            pltpu.VMEM((L,), jnp.int32),
            pltpu.VMEM((L,), jnp.int32),
            pltpu.VMEM((L,), jnp.float32),
            pltpu.SMEM((2,), jnp.int32),
            pltpu.SemaphoreType.DMA,
            pltpu.SemaphoreType.DMA,
        ],
    )
    return kern(t2d, p2d)


def _native_view(x):
    return x.reshape(B, T // 128, 128, 2).transpose(0, 1, 3, 2).reshape(B, ROW_F)


def kernel(y_true_affinity, y_pred_affinity):
    t2d = _native_view(y_true_affinity)
    p2d = _native_view(y_pred_affinity)
    parts = _run(t2d, p2d)
    packed = jnp.sum(parts[:, :3], axis=0).astype(jnp.float64)
    n_delta = jnp.asarray(float(B), jnp.float64)
    return (packed[0], n_delta, packed[1], packed[2])
